# baseline (speedup 1.0000x reference)
import functools

import jax
import jax.numpy as jnp
from jax import lax
from jax.experimental import pallas as pl
from jax.experimental.pallas import tpu as pltpu
import jax.experimental.pallas.tpu_sc as plsc

H = 1024
F = 2048
E = 8
N = 4096
K = 2
PAIRS = N * K
BLK = 256
CAP = PAIRS + E * BLK
NBLK = CAP // BLK
NW = 32
SLOTS_W = CAP // NW
GCH = 32
TOK_W = N // NW
CCH = 16


def _router_body(x_ref, wr_ref, br_ref, pos_ref, gates_ref, meta_ref):
    xv = x_ref[...]
    wr = wr_ref[...]
    logits = lax.dot_general(xv, wr, (((1,), (1,)), ((), ())),
                             preferred_element_type=jnp.float32)
    logits = logits + br_ref[...]
    iota8 = lax.broadcasted_iota(jnp.int32, (N, E), 1)
    m1 = jnp.max(logits, axis=1, keepdims=True)
    i1 = jnp.min(jnp.where(logits == m1, iota8, E), axis=1)
    masked = jnp.where(iota8 == i1[:, None], jnp.float32(-1e30), logits)
    m2 = jnp.max(masked, axis=1, keepdims=True)
    i2 = jnp.min(jnp.where(masked == m2, iota8, E), axis=1)
    g1 = 1.0 / (1.0 + jnp.exp(m2[:, 0] - m1[:, 0]))
    g2 = 1.0 - g1
    gates_ref[...] = jnp.concatenate(
        [g1.reshape(N // 128, 128), g2.reshape(N // 128, 128)], axis=0)

    pair_e = jnp.concatenate(
        [i1.reshape(N // 128, 128), i2.reshape(N // 128, 128)], axis=0)
    PR = PAIRS // 128

    offs, ends = [], []
    run = jnp.int32(0)
    for e in range(E):
        ce = jnp.sum((pair_e == e).astype(jnp.float32)).astype(jnp.int32)
        offs.append(run)
        run = run + ((ce + BLK - 1) // BLK) * BLK
        ends.append(run)

    ri = lax.broadcasted_iota(jnp.int32, (128, 128), 0)
    ci = lax.broadcasted_iota(jnp.int32, (128, 128), 1)
    U = (ri <= ci).astype(jnp.float32)
    rA = lax.broadcasted_iota(jnp.int32, (PR, PR), 0)
    cA = lax.broadcasted_iota(jnp.int32, (PR, PR), 1)
    A = (rA > cA).astype(jnp.float32)
    pos_acc = jnp.zeros((PR, 128), jnp.float32)
    for e in range(E):
        Me = (pair_e == e).astype(jnp.float32)
        R = lax.dot_general(Me, U, (((1,), (0,)), ((), ())),
                            preferred_element_type=jnp.float32)
        prev = lax.dot_general(A, R, (((1,), (0,)), ((), ())),
                               preferred_element_type=jnp.float32)
        rank = R + prev[:, 127:128] - Me
        pos_acc = pos_acc + Me * (offs[e].astype(jnp.float32) + rank)
    pos_ref[...] = pos_acc.astype(jnp.int32)

    bstart = lax.broadcasted_iota(jnp.int32, (1, 128), 1) * BLK
    be = jnp.zeros((1, 128), jnp.int32)
    for e in range(E):
        be = be + (bstart >= ends[e]).astype(jnp.int32)
    be = jnp.minimum(be, E - 1)
    act = (bstart < run).astype(jnp.int32)
    first = jnp.zeros((1, 128), jnp.float32)
    for e in range(E):
        nonempty = (ends[e] > offs[e]).astype(jnp.int32)
        first = first + ((bstart == offs[e]).astype(jnp.int32)
                         * nonempty).astype(jnp.float32)
    cnt = lax.dot_general(first, U, (((1,), (0,)), ((), ())),
                          preferred_element_type=jnp.float32)
    parity = jnp.bitwise_and(cnt.astype(jnp.int32) - 1, 1)
    nxt = jnp.full((1, 128), E, jnp.int32)
    for e in range(E):
        cond = jnp.logical_and(e > be, ends[e] > offs[e])
        nxt = jnp.minimum(nxt, jnp.where(cond, e, E))
    nxt = jnp.where(nxt == E, be, nxt)
    firsti = first.astype(jnp.int32)
    issue = firsti * (nxt != be).astype(jnp.int32)
    meta_ref[...] = jnp.concatenate(
        [be, act, parity, nxt, firsti, issue]
        + [jnp.zeros((1, 128), jnp.int32)] * 2, axis=0)


def _router_call(x2d, Wr, br2):
    return pl.pallas_call(
        _router_body,
        in_specs=[
            pl.BlockSpec((N, H), lambda: (0, 0)),
            pl.BlockSpec((E, H), lambda: (0, 0)),
            pl.BlockSpec((1, E), lambda: (0, 0)),
        ],
        out_specs=[
            pl.BlockSpec((PAIRS // 128, 128), lambda: (0, 0)),
            pl.BlockSpec((PAIRS // 128, 128), lambda: (0, 0)),
            pl.BlockSpec((8, 128), lambda: (0, 0)),
        ],
        out_shape=[
            jax.ShapeDtypeStruct((PAIRS // 128, 128), jnp.int32),
            jax.ShapeDtypeStruct((PAIRS // 128, 128), jnp.float32),
            jax.ShapeDtypeStruct((8, 128), jnp.int32),
        ],
    )(x2d, Wr, br2)


@functools.cache
def _sc_mesh():
    return plsc.VectorSubcoreMesh(core_axis_name="c", subcore_axis_name="s")


PCH = PAIRS // NW // GCH


@functools.cache
def _dispatch_kernel():
    return pl.kernel(
        _dispatch_body,
        mesh=_sc_mesh(),
        out_type=(jax.ShapeDtypeStruct((CAP, H), jnp.float32),
                  jax.ShapeDtypeStruct((CAP, 128), jnp.float32)),
        scratch_types=[
            pltpu.VMEM((PCH, GCH), jnp.int32),
            pltpu.VMEM((PCH, GCH), jnp.float32),
            pltpu.VMEM((GCH, H), jnp.float32),
            pltpu.VMEM((GCH, H), jnp.float32),
            pltpu.VMEM((GCH, 128), jnp.float32),
            pltpu.VMEM((GCH, 128), jnp.float32),
            pltpu.SemaphoreType.DMA,
            pltpu.SemaphoreType.DMA,
            pltpu.SemaphoreType.DMA,
            pltpu.SemaphoreType.DMA,
            pltpu.SemaphoreType.DMA,
            pltpu.SemaphoreType.DMA,
        ],
        compiler_params=pltpu.CompilerParams(needs_layout_passes=False),
    )


def _dispatch_body(pos_hbm, gat_hbm, x_hbm, xg_hbm, gsl_hbm,
                   posv, gatv, buf0, buf1, gb0, gb1,
                   sl0, sl1, ss0, ss1, sg0, sg1):
    wid = lax.axis_index("s") * 2 + lax.axis_index("c")
    t0 = jnp.bitwise_and(wid, 15) * (PCH * GCH)
    pltpu.sync_copy(pos_hbm.at[wid], posv)
    pltpu.sync_copy(gat_hbm.at[wid], gatv)
    bufs = (buf0, buf1)
    gbufs = (gb0, gb1)
    semsL = (sl0, sl1)
    semsS = (ss0, ss1)
    semsG = (sg0, sg1)
    descL = [None, None]
    descS = [None, None]
    descG = [None, None]
    descL[0] = pltpu.async_copy(x_hbm.at[pl.ds(t0, GCH)], buf0, sl0)
    zero16 = jnp.zeros((16,), jnp.int32)
    for c in range(PCH):
        b = c % 2
        nb = (c + 1) % 2
        descL[b].wait()
        if c + 1 < PCH:
            if descS[nb] is not None:
                descS[nb].wait()
            descL[nb] = pltpu.async_copy(
                x_hbm.at[pl.ds(t0 + (c + 1) * GCH, GCH)], bufs[nb], semsL[nb])
        descS[b] = pltpu.async_copy(bufs[b], xg_hbm.at[posv.at[c]], semsS[b])
        if descG[b] is not None:
            descG[b].wait()
        for j in range(GCH // 16):
            plsc.store_scatter(gbufs[b],
                               [lax.iota(jnp.int32, 16) + 16 * j, zero16],
                               gatv[c, pl.ds(16 * j, 16)])
        descG[b] = pltpu.async_copy(gbufs[b], gsl_hbm.at[posv.at[c]],
                                    semsG[b])
    for d in descS:
        d.wait()
    for d in descG:
        if d is not None:
            d.wait()


def _ffn_body(m_ref, xg_ref, w1_ref, b1_ref, w2_ref, b2_ref, g_ref, y_ref,
              w1r, w2r, s1, s2):
    i = pl.program_id(0)
    par = m_ref[2, i]

    @pl.when(i == 0)
    def _():
        pltpu.make_async_copy(w1_ref.at[m_ref[0, 0]], w1r.at[0],
                              s1.at[0]).start()
        pltpu.make_async_copy(w2_ref.at[m_ref[0, 0]], w2r.at[0],
                              s2.at[0]).start()

    @pl.when(m_ref[4, i] == 1)
    def _():
        @pl.when(m_ref[5, i] == 1)
        def _():
            nxt = m_ref[3, i]
            pltpu.make_async_copy(w1_ref.at[nxt], w1r.at[1 - par],
                                  s1.at[1 - par]).start()
            pltpu.make_async_copy(w2_ref.at[nxt], w2r.at[1 - par],
                                  s2.at[1 - par]).start()
        pltpu.make_async_copy(w1_ref.at[m_ref[0, i]], w1r.at[par],
                              s1.at[par]).wait()
        pltpu.make_async_copy(w2_ref.at[m_ref[0, i]], w2r.at[par],
                              s2.at[par]).wait()

    @pl.when(m_ref[1, i] == 1)
    def _():
        xb = xg_ref[...]
        h = lax.dot_general(xb, w1r[par], (((1,), (1,)), ((), ())),
                            preferred_element_type=jnp.float32)
        h = jnp.maximum(h + b1_ref[0], 0.0)
        y = lax.dot_general(h, w2r[par], (((1,), (1,)), ((), ())),
                            preferred_element_type=jnp.float32)
        y = y + b2_ref[0]
        y_ref[...] = y * g_ref[...][:, 0:1]


def _ffn_call(meta, xg, W1, b1r, W2, b2r, g3d):
    grid_spec = pltpu.PrefetchScalarGridSpec(
        num_scalar_prefetch=1,
        grid=(NBLK,),
        in_specs=[
            pl.BlockSpec((BLK, H), lambda i, m: (i, 0)),
            pl.BlockSpec(memory_space=pl.ANY),
            pl.BlockSpec((1, 1, F), lambda i, m: (m[0, i], 0, 0)),
            pl.BlockSpec(memory_space=pl.ANY),
            pl.BlockSpec((1, 1, H), lambda i, m: (m[0, i], 0, 0)),
            pl.BlockSpec((BLK, 128), lambda i, m: (i, 0)),
        ],
        out_specs=pl.BlockSpec((BLK, H), lambda i, m: (i, 0)),
        scratch_shapes=[
            pltpu.VMEM((2, F, H), jnp.float32),
            pltpu.VMEM((2, H, F), jnp.float32),
            pltpu.SemaphoreType.DMA((2,)),
            pltpu.SemaphoreType.DMA((2,)),
        ],
    )
    return pl.pallas_call(
        _ffn_body,
        grid_spec=grid_spec,
        out_shape=jax.ShapeDtypeStruct((CAP, H), jnp.float32),
        compiler_params=pltpu.CompilerParams(
            dimension_semantics=("arbitrary",)),
    )(meta, xg, W1, b1r, W2, b2r, g3d)


@functools.cache
def _combine_kernel():
    return pl.kernel(
        _combine_body,
        mesh=_sc_mesh(),
        out_type=jax.ShapeDtypeStruct((N, H), jnp.float32),
        scratch_types=[
            pltpu.VMEM((TOK_W,), jnp.int32),
            pltpu.VMEM((TOK_W,), jnp.int32),
            pltpu.VMEM((CCH, H), jnp.float32),
            pltpu.VMEM((CCH, H), jnp.float32),
            pltpu.VMEM((CCH, H), jnp.float32),
            pltpu.VMEM((CCH, H), jnp.float32),
            pltpu.SemaphoreType.DMA,
            pltpu.SemaphoreType.DMA,
            pltpu.SemaphoreType.DMA,
            pltpu.SemaphoreType.DMA,
            pltpu.SemaphoreType.DMA,
            pltpu.SemaphoreType.DMA,
        ],
        compiler_params=pltpu.CompilerParams(needs_layout_passes=False),
    )


def _combine_body(pp_hbm, y_hbm, out_hbm, p0v, p1v,
                  a0, b0, a1, b1, sa0, sb0, sa1, sb1, so0, so1):
    wid = lax.axis_index("s") * 2 + lax.axis_index("c")
    base = wid * TOK_W
    pltpu.sync_copy(pp_hbm.at[pl.ds(base, TOK_W)], p0v)
    pltpu.sync_copy(pp_hbm.at[pl.ds(N + base, TOK_W)], p1v)
    abufs = (a0, a1)
    bbufs = (b0, b1)
    semsA = (sa0, sa1)
    semsB = (sb0, sb1)
    semsO = (so0, so1)
    NCH = TOK_W // CCH
    descA = [None, None]
    descB = [None, None]
    descO = [None, None]

    def _gathers(c, b):
        descA[b] = pltpu.async_copy(
            y_hbm.at[p0v.at[pl.ds(c * CCH, CCH)]], abufs[b], semsA[b])
        descB[b] = pltpu.async_copy(
            y_hbm.at[p1v.at[pl.ds(c * CCH, CCH)]], bbufs[b], semsB[b])

    _gathers(0, 0)
    for c in range(NCH):
        b = c % 2
        nb = (c + 1) % 2
        if c + 1 < NCH:
            if descO[nb] is not None:
                descO[nb].wait()
            _gathers(c + 1, nb)
        descA[b].wait()
        descB[b].wait()
        bufa = abufs[b]
        bufb = bbufs[b]

        @plsc.parallel_loop(0, CCH * (H // 16), unroll=8)
        def _add(j):
            r = j // (H // 16)
            k = j % (H // 16)
            av = bufa[r, pl.ds(k * 16, 16)]
            bv = bufb[r, pl.ds(k * 16, 16)]
            bufa[r, pl.ds(k * 16, 16)] = av + bv
        descO[b] = pltpu.async_copy(
            bufa, out_hbm.at[pl.ds(base + c * CCH, CCH)], semsO[b])
    for d in descO:
        if d is not None:
            d.wait()


def kernel(x, Wr, br, W1, b1, W2, b2):
    x2d = x.reshape(N, H)
    pos_m, gates_m, meta = _router_call(x2d, Wr, br.reshape(1, E))
    pos_pairs = pos_m.reshape(-1)
    gates_pairs = gates_m.reshape(-1)
    xg, gsl = _dispatch_kernel()(pos_pairs.reshape(NW, PCH, GCH),
                                 gates_pairs.reshape(NW, PCH, GCH), x2d)
    y = _ffn_call(meta, xg, W1, b1.reshape(E, 1, F), W2, b2.reshape(E, 1, H),
                  gsl)
    out2d = _combine_kernel()(pos_pairs, y)
    return out2d.reshape(x.shape)


# speedup vs baseline: 1.0230x; 1.0230x over previous
import functools

import jax
import jax.numpy as jnp
from jax import lax
from jax.experimental import pallas as pl
from jax.experimental.pallas import tpu as pltpu
import jax.experimental.pallas.tpu_sc as plsc

H = 1024
F = 2048
E = 8
N = 4096
K = 2
PAIRS = N * K
BLK = 256
CAP = PAIRS + E * BLK
NBLK = CAP // BLK
NW = 32
SLOTS_W = CAP // NW
GCH = 32
TOK_W = N // NW
CCH = 16


def _router_body(x_ref, wr_ref, br_ref, pos_ref, gates_ref, meta_ref):
    xv = x_ref[...]
    wr = wr_ref[...]
    logits = lax.dot_general(xv, wr, (((1,), (1,)), ((), ())),
                             preferred_element_type=jnp.float32)
    logits = logits + br_ref[...]
    iota8 = lax.broadcasted_iota(jnp.int32, (N, E), 1)
    m1 = jnp.max(logits, axis=1, keepdims=True)
    i1 = jnp.min(jnp.where(logits == m1, iota8, E), axis=1)
    masked = jnp.where(iota8 == i1[:, None], jnp.float32(-1e30), logits)
    m2 = jnp.max(masked, axis=1, keepdims=True)
    i2 = jnp.min(jnp.where(masked == m2, iota8, E), axis=1)
    g1 = 1.0 / (1.0 + jnp.exp(m2[:, 0] - m1[:, 0]))
    g2 = 1.0 - g1
    gates_ref[...] = jnp.concatenate(
        [g1.reshape(N // 128, 128), g2.reshape(N // 128, 128)], axis=0)

    pair_e = jnp.concatenate(
        [i1.reshape(N // 128, 128), i2.reshape(N // 128, 128)], axis=0)
    PR = PAIRS // 128

    offs, ends = [], []
    run = jnp.int32(0)
    for e in range(E):
        ce = jnp.sum((pair_e == e).astype(jnp.float32)).astype(jnp.int32)
        offs.append(run)
        run = run + ((ce + BLK - 1) // BLK) * BLK
        ends.append(run)

    ri = lax.broadcasted_iota(jnp.int32, (128, 128), 0)
    ci = lax.broadcasted_iota(jnp.int32, (128, 128), 1)
    U = (ri <= ci).astype(jnp.float32)
    rA = lax.broadcasted_iota(jnp.int32, (PR, PR), 0)
    cA = lax.broadcasted_iota(jnp.int32, (PR, PR), 1)
    A = (rA > cA).astype(jnp.float32)
    pos_acc = jnp.zeros((PR, 128), jnp.float32)
    for e in range(E):
        Me = (pair_e == e).astype(jnp.float32)
        R = lax.dot_general(Me, U, (((1,), (0,)), ((), ())),
                            preferred_element_type=jnp.float32)
        prev = lax.dot_general(A, R, (((1,), (0,)), ((), ())),
                               preferred_element_type=jnp.float32)
        rank = R + prev[:, 127:128] - Me
        pos_acc = pos_acc + Me * (offs[e].astype(jnp.float32) + rank)
    pos_ref[...] = pos_acc.astype(jnp.int32)

    bstart = lax.broadcasted_iota(jnp.int32, (1, 128), 1) * BLK
    be = jnp.zeros((1, 128), jnp.int32)
    for e in range(E):
        be = be + (bstart >= ends[e]).astype(jnp.int32)
    be = jnp.minimum(be, E - 1)
    act = (bstart < run).astype(jnp.int32)
    first = jnp.zeros((1, 128), jnp.float32)
    for e in range(E):
        nonempty = (ends[e] > offs[e]).astype(jnp.int32)
        first = first + ((bstart == offs[e]).astype(jnp.int32)
                         * nonempty).astype(jnp.float32)
    cnt = lax.dot_general(first, U, (((1,), (0,)), ((), ())),
                          preferred_element_type=jnp.float32)
    parity = jnp.bitwise_and(cnt.astype(jnp.int32) - 1, 1)
    nxt = jnp.full((1, 128), E, jnp.int32)
    for e in range(E):
        cond = jnp.logical_and(e > be, ends[e] > offs[e])
        nxt = jnp.minimum(nxt, jnp.where(cond, e, E))
    nxt = jnp.where(nxt == E, be, nxt)
    firsti = first.astype(jnp.int32)
    issue = firsti * (nxt != be).astype(jnp.int32)
    biota = lax.broadcasted_iota(jnp.int32, (1, 128), 1)
    blk_idx = jnp.minimum(biota, jnp.maximum(run // BLK - 1, 0))
    meta_ref[...] = jnp.concatenate(
        [be, act, parity, nxt, firsti, issue, blk_idx,
         jnp.zeros((1, 128), jnp.int32)], axis=0)


def _router_call(x2d, Wr, br2):
    return pl.pallas_call(
        _router_body,
        in_specs=[
            pl.BlockSpec((N, H), lambda: (0, 0)),
            pl.BlockSpec((E, H), lambda: (0, 0)),
            pl.BlockSpec((1, E), lambda: (0, 0)),
        ],
        out_specs=[
            pl.BlockSpec((PAIRS // 128, 128), lambda: (0, 0)),
            pl.BlockSpec((PAIRS // 128, 128), lambda: (0, 0)),
            pl.BlockSpec((8, 128), lambda: (0, 0)),
        ],
        out_shape=[
            jax.ShapeDtypeStruct((PAIRS // 128, 128), jnp.int32),
            jax.ShapeDtypeStruct((PAIRS // 128, 128), jnp.float32),
            jax.ShapeDtypeStruct((8, 128), jnp.int32),
        ],
    )(x2d, Wr, br2)


@functools.cache
def _sc_mesh():
    return plsc.VectorSubcoreMesh(core_axis_name="c", subcore_axis_name="s")


PCH = PAIRS // NW // GCH


@functools.cache
def _dispatch_kernel():
    return pl.kernel(
        _dispatch_body,
        mesh=_sc_mesh(),
        out_type=(jax.ShapeDtypeStruct((CAP, H), jnp.float32),
                  jax.ShapeDtypeStruct((CAP, 128), jnp.float32)),
        scratch_types=[
            pltpu.VMEM((PCH, GCH), jnp.int32),
            pltpu.VMEM((PCH, GCH), jnp.float32),
            pltpu.VMEM((GCH, H), jnp.float32),
            pltpu.VMEM((GCH, H), jnp.float32),
            pltpu.VMEM((GCH, 128), jnp.float32),
            pltpu.VMEM((GCH, 128), jnp.float32),
            pltpu.SemaphoreType.DMA,
            pltpu.SemaphoreType.DMA,
            pltpu.SemaphoreType.DMA,
            pltpu.SemaphoreType.DMA,
            pltpu.SemaphoreType.DMA,
            pltpu.SemaphoreType.DMA,
        ],
        compiler_params=pltpu.CompilerParams(needs_layout_passes=False),
    )


def _dispatch_body(pos_hbm, gat_hbm, x_hbm, xg_hbm, gsl_hbm,
                   posv, gatv, buf0, buf1, gb0, gb1,
                   sl0, sl1, ss0, ss1, sg0, sg1):
    wid = lax.axis_index("s") * 2 + lax.axis_index("c")
    t0 = jnp.bitwise_and(wid, 15) * (PCH * GCH)
    pltpu.sync_copy(pos_hbm.at[wid], posv)
    pltpu.sync_copy(gat_hbm.at[wid], gatv)
    bufs = (buf0, buf1)
    gbufs = (gb0, gb1)
    semsL = (sl0, sl1)
    semsS = (ss0, ss1)
    semsG = (sg0, sg1)
    descL = [None, None]
    descS = [None, None]
    descG = [None, None]
    descL[0] = pltpu.async_copy(x_hbm.at[pl.ds(t0, GCH)], buf0, sl0)
    zero16 = jnp.zeros((16,), jnp.int32)
    for c in range(PCH):
        b = c % 2
        nb = (c + 1) % 2
        descL[b].wait()
        if c + 1 < PCH:
            if descS[nb] is not None:
                descS[nb].wait()
            descL[nb] = pltpu.async_copy(
                x_hbm.at[pl.ds(t0 + (c + 1) * GCH, GCH)], bufs[nb], semsL[nb])
        descS[b] = pltpu.async_copy(bufs[b], xg_hbm.at[posv.at[c]], semsS[b])
        if descG[b] is not None:
            descG[b].wait()
        for j in range(GCH // 16):
            plsc.store_scatter(gbufs[b],
                               [lax.iota(jnp.int32, 16) + 16 * j, zero16],
                               gatv[c, pl.ds(16 * j, 16)])
        descG[b] = pltpu.async_copy(gbufs[b], gsl_hbm.at[posv.at[c]],
                                    semsG[b])
    for d in descS:
        d.wait()
    for d in descG:
        if d is not None:
            d.wait()


def _ffn_body(m_ref, xg_ref, w1_ref, b1_ref, w2_ref, b2_ref, g_ref, y_ref,
              w1r, w2r, s1, s2):
    i = pl.program_id(0)
    par = m_ref[2, i]

    @pl.when(i == 0)
    def _():
        pltpu.make_async_copy(w1_ref.at[m_ref[0, 0]], w1r.at[0],
                              s1.at[0]).start()
        pltpu.make_async_copy(w2_ref.at[m_ref[0, 0]], w2r.at[0],
                              s2.at[0]).start()

    @pl.when(m_ref[4, i] == 1)
    def _():
        @pl.when(m_ref[5, i] == 1)
        def _():
            nxt = m_ref[3, i]
            pltpu.make_async_copy(w1_ref.at[nxt], w1r.at[1 - par],
                                  s1.at[1 - par]).start()
            pltpu.make_async_copy(w2_ref.at[nxt], w2r.at[1 - par],
                                  s2.at[1 - par]).start()
        pltpu.make_async_copy(w1_ref.at[m_ref[0, i]], w1r.at[par],
                              s1.at[par]).wait()
        pltpu.make_async_copy(w2_ref.at[m_ref[0, i]], w2r.at[par],
                              s2.at[par]).wait()

    @pl.when(m_ref[1, i] == 1)
    def _():
        xb = xg_ref[...]
        h = lax.dot_general(xb, w1r[par], (((1,), (1,)), ((), ())),
                            preferred_element_type=jnp.float32)
        h = jnp.maximum(h + b1_ref[0], 0.0)
        y = lax.dot_general(h, w2r[par], (((1,), (1,)), ((), ())),
                            preferred_element_type=jnp.float32)
        y = y + b2_ref[0]
        y_ref[...] = y * g_ref[...][:, 0:1]


def _ffn_call(meta, xg, W1, b1r, W2, b2r, g3d):
    grid_spec = pltpu.PrefetchScalarGridSpec(
        num_scalar_prefetch=1,
        grid=(NBLK,),
        in_specs=[
            pl.BlockSpec((BLK, H), lambda i, m: (m[6, i], 0)),
            pl.BlockSpec(memory_space=pl.ANY),
            pl.BlockSpec((1, 1, F), lambda i, m: (m[0, i], 0, 0)),
            pl.BlockSpec(memory_space=pl.ANY),
            pl.BlockSpec((1, 1, H), lambda i, m: (m[0, i], 0, 0)),
            pl.BlockSpec((BLK, 128), lambda i, m: (m[6, i], 0)),
        ],
        out_specs=pl.BlockSpec((BLK, H), lambda i, m: (m[6, i], 0)),
        scratch_shapes=[
            pltpu.VMEM((2, F, H), jnp.float32),
            pltpu.VMEM((2, H, F), jnp.float32),
            pltpu.SemaphoreType.DMA((2,)),
            pltpu.SemaphoreType.DMA((2,)),
        ],
    )
    return pl.pallas_call(
        _ffn_body,
        grid_spec=grid_spec,
        out_shape=jax.ShapeDtypeStruct((CAP, H), jnp.float32),
        compiler_params=pltpu.CompilerParams(
            dimension_semantics=("arbitrary",)),
    )(meta, xg, W1, b1r, W2, b2r, g3d)


@functools.cache
def _combine_kernel():
    return pl.kernel(
        _combine_body,
        mesh=_sc_mesh(),
        out_type=jax.ShapeDtypeStruct((N, H), jnp.float32),
        scratch_types=[
            pltpu.VMEM((TOK_W,), jnp.int32),
            pltpu.VMEM((TOK_W,), jnp.int32),
            pltpu.VMEM((CCH, H), jnp.float32),
            pltpu.VMEM((CCH, H), jnp.float32),
            pltpu.VMEM((CCH, H), jnp.float32),
            pltpu.VMEM((CCH, H), jnp.float32),
            pltpu.SemaphoreType.DMA,
            pltpu.SemaphoreType.DMA,
            pltpu.SemaphoreType.DMA,
            pltpu.SemaphoreType.DMA,
            pltpu.SemaphoreType.DMA,
            pltpu.SemaphoreType.DMA,
        ],
        compiler_params=pltpu.CompilerParams(needs_layout_passes=False),
    )


def _combine_body(pp_hbm, y_hbm, out_hbm, p0v, p1v,
                  a0, b0, a1, b1, sa0, sb0, sa1, sb1, so0, so1):
    wid = lax.axis_index("s") * 2 + lax.axis_index("c")
    base = wid * TOK_W
    pltpu.sync_copy(pp_hbm.at[pl.ds(base, TOK_W)], p0v)
    pltpu.sync_copy(pp_hbm.at[pl.ds(N + base, TOK_W)], p1v)
    abufs = (a0, a1)
    bbufs = (b0, b1)
    semsA = (sa0, sa1)
    semsB = (sb0, sb1)
    semsO = (so0, so1)
    NCH = TOK_W // CCH
    descA = [None, None]
    descB = [None, None]
    descO = [None, None]

    def _gathers(c, b):
        descA[b] = pltpu.async_copy(
            y_hbm.at[p0v.at[pl.ds(c * CCH, CCH)]], abufs[b], semsA[b])
        descB[b] = pltpu.async_copy(
            y_hbm.at[p1v.at[pl.ds(c * CCH, CCH)]], bbufs[b], semsB[b])

    _gathers(0, 0)
    for c in range(NCH):
        b = c % 2
        nb = (c + 1) % 2
        if c + 1 < NCH:
            if descO[nb] is not None:
                descO[nb].wait()
            _gathers(c + 1, nb)
        descA[b].wait()
        descB[b].wait()
        bufa = abufs[b]
        bufb = bbufs[b]

        @plsc.parallel_loop(0, CCH * (H // 16), unroll=8)
        def _add(j):
            r = j // (H // 16)
            k = j % (H // 16)
            av = bufa[r, pl.ds(k * 16, 16)]
            bv = bufb[r, pl.ds(k * 16, 16)]
            bufa[r, pl.ds(k * 16, 16)] = av + bv
        descO[b] = pltpu.async_copy(
            bufa, out_hbm.at[pl.ds(base + c * CCH, CCH)], semsO[b])
    for d in descO:
        if d is not None:
            d.wait()


def kernel(x, Wr, br, W1, b1, W2, b2):
    x2d = x.reshape(N, H)
    pos_m, gates_m, meta = _router_call(x2d, Wr, br.reshape(1, E))
    pos_pairs = pos_m.reshape(-1)
    gates_pairs = gates_m.reshape(-1)
    xg, gsl = _dispatch_kernel()(pos_pairs.reshape(NW, PCH, GCH),
                                 gates_pairs.reshape(NW, PCH, GCH), x2d)
    y = _ffn_call(meta, xg, W1, b1.reshape(E, 1, F), W2, b2.reshape(E, 1, H),
                  gsl)
    out2d = _combine_kernel()(pos_pairs, y)
    return out2d.reshape(x.shape)
